# baseline (speedup 1.0000x reference)
import functools

import numpy as np
import jax
import jax.numpy as jnp
from jax import lax
from jax.experimental import pallas as pl
from jax.experimental.pallas import tpu as pltpu
from jax.experimental.pallas import tpu_sc as plsc

N = 10000
DEG = 16
IN_DIM = 256
H1 = 16
H2 = 16
NB = 9
NBINS = H1 * NB
SPEC = 2 * NBINS
SPEC_PAD = 384
OUT_DIM = 256

NC = 2
NS = 16
NW = NC * NS
NPAD = 10240
CORE_NODES = (208, 432)
CHUNK_D = 8
CHUNK_IDX = CHUNK_D * DEG
MAX_IDX_PER_W = max(CORE_NODES) * DEG
SRC_PAD_LEN = ((16 * CORE_NODES[0] + 15 * CORE_NODES[1]) + max(CORE_NODES)) * DEG


def _build_constants():
    j = np.arange(H1)
    k = np.arange(H2)
    u = np.arange(H1)
    v = np.arange(NB)
    uj = np.outer(u, j)
    vk = np.outer(v, k)
    theta = 2.0 * np.pi * (uj[:, None, :, None] + vk[None, :, None, :]) / float(H1)
    theta = theta.transpose(2, 3, 0, 1).reshape(H1 * H2, NBINS)
    kf = np.concatenate([np.cos(theta), -np.sin(theta)], axis=1)
    w = np.ones(NB)
    w[1:8] = 2.0
    wp = np.repeat(w[None, :], H1, axis=0).reshape(-1)
    ikr = (wp[:, None] * np.cos(theta.T)) / float(H1 * H2)
    iki = (-wp[:, None] * np.sin(theta.T)) / float(H1 * H2)
    ik = np.concatenate([ikr, iki], axis=0)
    kf_pad = np.zeros((H1 * H2, SPEC_PAD), np.float32)
    kf_pad[:, :SPEC] = kf
    return kf_pad, ik.astype(np.float32)


_KF, _IK = _build_constants()


def _mm2_body(prec1, prec2, x_ref, w1_ref, b1_ref, w2_ref, b2_ref, o_ref):
    p = jnp.dot(x_ref[...], w1_ref[...], preferred_element_type=jnp.float32,
                precision=prec1)
    p = p + b1_ref[...]
    o = jnp.dot(p, w2_ref[...], preferred_element_type=jnp.float32,
                precision=prec2)
    o_ref[...] = o + b2_ref[...]


def _mm2(x, w1, b1, w2, b2, rows_per_block, out_rows, prec1, prec2):
    k1 = w1.shape[0]
    k2 = w2.shape[0]
    m = w2.shape[1]
    grid = (out_rows // rows_per_block,)
    return pl.pallas_call(
        functools.partial(_mm2_body, prec1, prec2),
        grid=grid,
        in_specs=[
            pl.BlockSpec((rows_per_block, k1), lambda i: (i, 0)),
            pl.BlockSpec((k1, k2), lambda i: (0, 0)),
            pl.BlockSpec((1, k2), lambda i: (0, 0)),
            pl.BlockSpec((k2, m), lambda i: (0, 0)),
            pl.BlockSpec((1, m), lambda i: (0, 0)),
        ],
        out_specs=pl.BlockSpec((rows_per_block, m), lambda i: (i, 0)),
        out_shape=jax.ShapeDtypeStruct((out_rows, m), jnp.float32),
    )(x, w1, b1, w2, b2)


def _sc_neighbor_product(y, src_pad):
    mesh = plsc.VectorSubcoreMesh(core_axis_name="c", subcore_axis_name="s")

    @functools.partial(
        pl.kernel,
        mesh=mesh,
        out_type=jax.ShapeDtypeStruct((NPAD, SPEC), jnp.float32),
        scratch_types=[
            pltpu.VMEM((MAX_IDX_PER_W,), jnp.int32),
            pltpu.VMEM((2, CHUNK_IDX, SPEC_PAD), jnp.float32),
            pltpu.VMEM((2, CHUNK_D, SPEC), jnp.float32),
            pltpu.SemaphoreType.DMA,
            pltpu.SemaphoreType.DMA,
            pltpu.SemaphoreType.DMA,
            pltpu.SemaphoreType.DMA,
        ],
    )
    def sc_prod(y_hbm, src_hbm, z_hbm, idx_v, rows_v, z_v,
                sem0, sem1, osem0, osem1):
        cid = lax.axis_index("c")
        sid = lax.axis_index("s")
        a, bnodes = CORE_NODES
        node_start = jnp.where(cid == 0, sid * a, NS * a + sid * bnodes)
        nchunks_w = jnp.where(cid == 0, a // CHUNK_D, bnodes // CHUNK_D)
        ibase = node_start * DEG
        pltpu.sync_copy(src_hbm.at[pl.ds(ibase, MAX_IDX_PER_W)], idx_v)

        def start(chunk, buf):
            sem = sem0 if buf == 0 else sem1
            pltpu.make_async_copy(
                y_hbm.at[idx_v.at[pl.ds(chunk * CHUNK_IDX, CHUNK_IDX)]],
                rows_v.at[buf],
                sem,
            ).start()

        def wait(chunk, buf):
            sem = sem0 if buf == 0 else sem1
            pltpu.make_async_copy(
                y_hbm.at[idx_v.at[pl.ds(chunk * CHUNK_IDX, CHUNK_IDX)]],
                rows_v.at[buf],
                sem,
            ).wait()

        def owait(chunk, buf):
            sem = osem0 if buf == 0 else osem1
            node0 = node_start + chunk * CHUNK_D
            pltpu.make_async_copy(
                z_v.at[buf], z_hbm.at[pl.ds(node0, CHUNK_D)], sem).wait()

        start(0, 0)
        start(1, 1)

        def outer(c2, carry):
            for b in range(2):
                chunk = c2 * 2 + b
                wait(chunk, b)

                @pl.when(chunk >= 2)
                def _():
                    owait(chunk - 2, b)

                def dbody(dd, c):
                    row0 = dd * DEG
                    for j in range(NBINS // 16):
                        sl_r = pl.ds(j * 16, 16)
                        sl_i = pl.ds(NBINS + j * 16, 16)
                        ar = rows_v[b, row0, sl_r]
                        ai = rows_v[b, row0, sl_i]
                        for k in range(1, DEG):
                            br = rows_v[b, row0 + k, sl_r]
                            bi = rows_v[b, row0 + k, sl_i]
                            nr = ar * br - ai * bi
                            ni = ar * bi + ai * br
                            ar = nr
                            ai = ni
                        z_v[b, dd, sl_r] = ar
                        z_v[b, dd, sl_i] = ai
                    return c

                lax.fori_loop(0, CHUNK_D, dbody, 0)
                node0 = node_start + chunk * CHUNK_D
                pltpu.make_async_copy(
                    z_v.at[b], z_hbm.at[pl.ds(node0, CHUNK_D)],
                    osem0 if b == 0 else osem1).start()

                @pl.when(chunk + 2 < nchunks_w)
                def _():
                    start(chunk + 2, b)
            return carry

        lax.fori_loop(0, nchunks_w // 2, outer, 0)
        owait(nchunks_w - 2, 0)
        owait(nchunks_w - 1, 1)

    return sc_prod(y, src_pad)


def kernel(feature, edge_index, W_aff, b_aff, W_mlp, b_mlp):
    kf = jnp.asarray(_KF)
    ik = jnp.asarray(_IK)
    src = edge_index[0].astype(jnp.int32)
    src_pad = jnp.concatenate(
        [src, jnp.zeros((SRC_PAD_LEN - N * DEG,), jnp.int32)])

    y = _mm2(feature, W_aff, b_aff[None, :], kf,
             jnp.zeros((1, SPEC_PAD), jnp.float32),
             rows_per_block=2048, out_rows=NPAD,
             prec1=lax.Precision.DEFAULT, prec2=lax.Precision.HIGHEST)

    z = _sc_neighbor_product(y, src_pad)

    out = _mm2(z, ik, jnp.zeros((1, OUT_DIM), jnp.float32),
               W_mlp, b_mlp[None, :], rows_per_block=2000, out_rows=N,
               prec1=lax.Precision.HIGHEST, prec2=lax.Precision.DEFAULT)
    return out


# speedup vs baseline: 1.1321x; 1.1321x over previous
import functools

import numpy as np
import jax
import jax.numpy as jnp
from jax import lax
from jax.experimental import pallas as pl
from jax.experimental.pallas import tpu as pltpu
from jax.experimental.pallas import tpu_sc as plsc

N = 10000
DEG = 16
IN_DIM = 256
H1 = 16
H2 = 16
NB = 9
NBINS = H1 * NB
SPEC = 2 * NBINS
SPEC_PAD = 384
OUT_DIM = 256

NC = 2
NS = 16
NW = NC * NS
NPAD = 10240
CORE_NODES = (432, 208)
CHUNK_D = 8
CHUNK_IDX = CHUNK_D * DEG
MAX_IDX_PER_W = max(CORE_NODES) * DEG
SRC_PAD_LEN = ((16 * CORE_NODES[0] + 15 * CORE_NODES[1]) + max(CORE_NODES)) * DEG


def _build_constants():
    j = np.arange(H1)
    k = np.arange(H2)
    u = np.arange(H1)
    v = np.arange(NB)
    uj = np.outer(u, j)
    vk = np.outer(v, k)
    theta = 2.0 * np.pi * (uj[:, None, :, None] + vk[None, :, None, :]) / float(H1)
    theta = theta.transpose(2, 3, 0, 1).reshape(H1 * H2, NBINS)
    kf = np.concatenate([np.cos(theta), -np.sin(theta)], axis=1)
    w = np.ones(NB)
    w[1:8] = 2.0
    wp = np.repeat(w[None, :], H1, axis=0).reshape(-1)
    ikr = (wp[:, None] * np.cos(theta.T)) / float(H1 * H2)
    iki = (-wp[:, None] * np.sin(theta.T)) / float(H1 * H2)
    ik = np.concatenate([ikr, iki], axis=0)
    kf_pad = np.zeros((H1 * H2, SPEC_PAD), np.float32)
    kf_pad[:, :SPEC] = kf
    return kf_pad, ik.astype(np.float32)


_KF, _IK = _build_constants()


def _mm2_body(prec1, prec2, x_ref, w1_ref, b1_ref, w2_ref, b2_ref, o_ref):
    p = jnp.dot(x_ref[...], w1_ref[...], preferred_element_type=jnp.float32,
                precision=prec1)
    p = p + b1_ref[...]
    o = jnp.dot(p, w2_ref[...], preferred_element_type=jnp.float32,
                precision=prec2)
    o_ref[...] = o + b2_ref[...]


def _mm2(x, w1, b1, w2, b2, rows_per_block, out_rows, prec1, prec2):
    k1 = w1.shape[0]
    k2 = w2.shape[0]
    m = w2.shape[1]
    grid = (out_rows // rows_per_block,)
    return pl.pallas_call(
        functools.partial(_mm2_body, prec1, prec2),
        grid=grid,
        in_specs=[
            pl.BlockSpec((rows_per_block, k1), lambda i: (i, 0)),
            pl.BlockSpec((k1, k2), lambda i: (0, 0)),
            pl.BlockSpec((1, k2), lambda i: (0, 0)),
            pl.BlockSpec((k2, m), lambda i: (0, 0)),
            pl.BlockSpec((1, m), lambda i: (0, 0)),
        ],
        out_specs=pl.BlockSpec((rows_per_block, m), lambda i: (i, 0)),
        out_shape=jax.ShapeDtypeStruct((out_rows, m), jnp.float32),
    )(x, w1, b1, w2, b2)


def _sc_neighbor_product(y, src_pad):
    mesh = plsc.VectorSubcoreMesh(core_axis_name="c", subcore_axis_name="s")

    @functools.partial(
        pl.kernel,
        mesh=mesh,
        out_type=jax.ShapeDtypeStruct((NPAD, SPEC), jnp.float32),
        scratch_types=[
            pltpu.VMEM((MAX_IDX_PER_W,), jnp.int32),
            pltpu.VMEM((2, CHUNK_IDX, SPEC_PAD), jnp.float32),
            pltpu.VMEM((2, CHUNK_D, SPEC), jnp.float32),
            pltpu.SemaphoreType.DMA,
            pltpu.SemaphoreType.DMA,
            pltpu.SemaphoreType.DMA,
            pltpu.SemaphoreType.DMA,
        ],
    )
    def sc_prod(y_hbm, src_hbm, z_hbm, idx_v, rows_v, z_v,
                sem0, sem1, osem0, osem1):
        cid = lax.axis_index("c")
        sid = lax.axis_index("s")
        a, bnodes = CORE_NODES
        node_start = jnp.where(cid == 0, sid * a, NS * a + sid * bnodes)
        nchunks_w = jnp.where(cid == 0, a // CHUNK_D, bnodes // CHUNK_D)
        ibase = node_start * DEG
        pltpu.sync_copy(src_hbm.at[pl.ds(ibase, MAX_IDX_PER_W)], idx_v)

        def start(chunk, buf):
            sem = sem0 if buf == 0 else sem1
            pltpu.make_async_copy(
                y_hbm.at[idx_v.at[pl.ds(chunk * CHUNK_IDX, CHUNK_IDX)]],
                rows_v.at[buf],
                sem,
            ).start()

        def wait(chunk, buf):
            sem = sem0 if buf == 0 else sem1
            pltpu.make_async_copy(
                y_hbm.at[idx_v.at[pl.ds(chunk * CHUNK_IDX, CHUNK_IDX)]],
                rows_v.at[buf],
                sem,
            ).wait()

        def owait(chunk, buf):
            sem = osem0 if buf == 0 else osem1
            node0 = node_start + chunk * CHUNK_D
            pltpu.make_async_copy(
                z_v.at[buf], z_hbm.at[pl.ds(node0, CHUNK_D)], sem).wait()

        start(0, 0)
        start(1, 1)

        def outer(c2, carry):
            for b in range(2):
                chunk = c2 * 2 + b
                wait(chunk, b)

                @pl.when(chunk >= 2)
                def _():
                    owait(chunk - 2, b)

                def dbody(dd, c):
                    row0 = dd * DEG
                    for j in range(NBINS // 16):
                        sl_r = pl.ds(j * 16, 16)
                        sl_i = pl.ds(NBINS + j * 16, 16)
                        ar = rows_v[b, row0, sl_r]
                        ai = rows_v[b, row0, sl_i]
                        for k in range(1, DEG):
                            br = rows_v[b, row0 + k, sl_r]
                            bi = rows_v[b, row0 + k, sl_i]
                            nr = ar * br - ai * bi
                            ni = ar * bi + ai * br
                            ar = nr
                            ai = ni
                        z_v[b, dd, sl_r] = ar
                        z_v[b, dd, sl_i] = ai
                    return c

                lax.fori_loop(0, CHUNK_D, dbody, 0)
                node0 = node_start + chunk * CHUNK_D
                pltpu.make_async_copy(
                    z_v.at[b], z_hbm.at[pl.ds(node0, CHUNK_D)],
                    osem0 if b == 0 else osem1).start()

                @pl.when(chunk + 2 < nchunks_w)
                def _():
                    start(chunk + 2, b)
            return carry

        lax.fori_loop(0, nchunks_w // 2, outer, 0)
        owait(nchunks_w - 2, 0)
        owait(nchunks_w - 1, 1)

    return sc_prod(y, src_pad)


def kernel(feature, edge_index, W_aff, b_aff, W_mlp, b_mlp):
    kf = jnp.asarray(_KF)
    ik = jnp.asarray(_IK)
    src = edge_index[0].astype(jnp.int32)
    src_pad = jnp.concatenate(
        [src, jnp.zeros((SRC_PAD_LEN - N * DEG,), jnp.int32)])

    y = _mm2(feature, W_aff, b_aff[None, :], kf,
             jnp.zeros((1, SPEC_PAD), jnp.float32),
             rows_per_block=2048, out_rows=NPAD,
             prec1=lax.Precision.DEFAULT, prec2=lax.Precision.HIGHEST)

    z = _sc_neighbor_product(y, src_pad)

    out = _mm2(z, ik, jnp.zeros((1, OUT_DIM), jnp.float32),
               W_mlp, b_mlp[None, :], rows_per_block=2000, out_rows=N,
               prec1=lax.Precision.HIGHEST, prec2=lax.Precision.DEFAULT)
    return out
